# baseline (speedup 1.0000x reference)
import functools

import jax
import jax.numpy as jnp
from jax import lax
from jax.experimental import pallas as pl
from jax.experimental.pallas import tpu as pltpu
from jax.experimental.pallas import tpu_sc as plsc

_NUM_CORES = 1
_NUM_SUBCORES = 16
_LANES = 16


def _pooling_kernel(B, S, D):
    mesh = plsc.VectorSubcoreMesh(
        core_axis_name="c",
        subcore_axis_name="s",
        num_cores=_NUM_CORES,
        num_subcores=_NUM_SUBCORES,
    )
    rows_per_core = B // _NUM_CORES
    workers_per_row = _NUM_SUBCORES // rows_per_core
    chunk = S // workers_per_row
    iters = chunk // _LANES

    @functools.partial(
        pl.kernel,
        out_type=jax.ShapeDtypeStruct((B, D), jnp.float32),
        mesh=mesh,
        scratch_types=[
            pltpu.VMEM((chunk,), jnp.int32),
            pltpu.VMEM((_LANES,), jnp.int32),
            pltpu.VMEM((workers_per_row, _LANES), jnp.int32),
            pltpu.VMEM((1, D), jnp.float32),
            pltpu.VMEM_SHARED((_NUM_SUBCORES, _LANES), jnp.int32),
        ],
        compiler_params=pltpu.CompilerParams(
            needs_layout_passes=False,
            skip_device_barrier=True,
            use_tc_tiling_on_sc=False,
        ),
    )
    def body(ids_hbm, lhs_hbm, out_hbm, ids_v, acc_v, parts_v, row_v, shared):
        sid = lax.axis_index("s")
        cid = lax.axis_index("c")
        b = rows_per_core * cid + sid // workers_per_row
        part = sid % workers_per_row

        pltpu.sync_copy(ids_hbm.at[b, pl.ds(part * chunk, chunk)], ids_v)

        rev_pos = (
            jnp.int32(S - 1 - part * chunk) - lax.iota(jnp.int32, _LANES)
        )
        acc = jnp.full((_LANES,), jnp.iinfo(jnp.int32).min, jnp.int32)
        for i in range(iters):
            vals = ids_v[pl.ds(i * _LANES, _LANES)]
            keys = lax.shift_left(vals, jnp.int32(13)) + (
                rev_pos - jnp.int32(i * _LANES)
            )
            acc = jnp.maximum(acc, keys)

        acc_v[...] = acc
        pltpu.sync_copy(acc_v, shared.at[sid])
        plsc.subcore_barrier()

        @pl.when(part == 0)
        def _():
            base = (sid // workers_per_row) * workers_per_row
            pltpu.sync_copy(shared.at[pl.ds(base, workers_per_row)], parts_v)
            red = parts_v[0, :]
            for w in range(1, workers_per_row):
                red = jnp.maximum(red, parts_v[w, :])
            best = jnp.max(red)
            idx = jnp.int32(S - 1) - lax.bitwise_and(best, jnp.int32(S - 1))

            pltpu.sync_copy(lhs_hbm.at[b, pl.ds(idx, 1)], row_v)
            pltpu.sync_copy(row_v, out_hbm.at[pl.ds(b, 1)])

    return body


def kernel(input_ids, last_hidden_state):
    B, S, D = last_hidden_state.shape
    ids = input_ids.astype(jnp.int32)
    return _pooling_kernel(B, S, D)(ids, last_hidden_state)


# speedup vs baseline: 10.4817x; 10.4817x over previous
import functools

import jax
import jax.numpy as jnp
from jax import lax
from jax.experimental import pallas as pl
from jax.experimental.pallas import tpu as pltpu
from jax.experimental.pallas import tpu_sc as plsc

_NUM_CORES = 1
_NUM_SUBCORES = 16
_LANES = 16


def _pooling_kernel(B, S, D):
    mesh = plsc.VectorSubcoreMesh(
        core_axis_name="c",
        subcore_axis_name="s",
        num_cores=_NUM_CORES,
        num_subcores=_NUM_SUBCORES,
    )
    rows_per_core = B // _NUM_CORES
    workers_per_row = _NUM_SUBCORES // rows_per_core
    chunk = S // workers_per_row
    iters = chunk // _LANES

    @functools.partial(
        pl.kernel,
        out_type=jax.ShapeDtypeStruct((B, D), jnp.float32),
        mesh=mesh,
        scratch_types=[
            pltpu.VMEM((chunk,), jnp.int32),
            pltpu.VMEM((_LANES,), jnp.int32),
            pltpu.VMEM((workers_per_row, _LANES), jnp.int32),
            pltpu.VMEM((1, D), jnp.float32),
            pltpu.VMEM_SHARED((_NUM_SUBCORES, _LANES), jnp.int32),
            pltpu.SemaphoreType.DMA,
            pltpu.SemaphoreType.DMA,
        ],
        compiler_params=pltpu.CompilerParams(
            needs_layout_passes=False, skip_device_barrier=True
        ),
    )
    def body(
        ids_hbm, lhs_hbm, out_hbm, ids_v, acc_v, parts_v, row_v, shared,
        sem0, sem1,
    ):
        sid = lax.axis_index("s")
        cid = lax.axis_index("c")
        b = rows_per_core * cid + sid // workers_per_row
        part = sid % workers_per_row

        half = chunk // 2
        cp0 = pltpu.async_copy(
            ids_hbm.at[b, pl.ds(part * chunk, half)],
            ids_v.at[pl.ds(0, half)],
            sem0,
        )
        cp1 = pltpu.async_copy(
            ids_hbm.at[b, pl.ds(part * chunk + half, half)],
            ids_v.at[pl.ds(half, half)],
            sem1,
        )

        rev_pos = (
            jnp.int32(S - 1 - part * chunk) - lax.iota(jnp.int32, _LANES)
        )
        acc = jnp.full((_LANES,), jnp.iinfo(jnp.int32).min, jnp.int32)
        cp0.wait()
        for i in range(iters // 2):
            vals = ids_v[pl.ds(i * _LANES, _LANES)]
            keys = lax.shift_left(vals, jnp.int32(13)) + (
                rev_pos - jnp.int32(i * _LANES)
            )
            acc = jnp.maximum(acc, keys)
        cp1.wait()
        for i in range(iters // 2, iters):
            vals = ids_v[pl.ds(i * _LANES, _LANES)]
            keys = lax.shift_left(vals, jnp.int32(13)) + (
                rev_pos - jnp.int32(i * _LANES)
            )
            acc = jnp.maximum(acc, keys)

        acc_v[...] = acc
        pltpu.sync_copy(acc_v, shared.at[sid])
        plsc.subcore_barrier()

        @pl.when(part == 0)
        def _():
            base = (sid // workers_per_row) * workers_per_row
            pltpu.sync_copy(shared.at[pl.ds(base, workers_per_row)], parts_v)
            red = parts_v[0, :]
            for w in range(1, workers_per_row):
                red = jnp.maximum(red, parts_v[w, :])
            best = jnp.max(red)
            idx = jnp.int32(S - 1) - lax.bitwise_and(best, jnp.int32(S - 1))

            pltpu.sync_copy(lhs_hbm.at[b, pl.ds(idx, 1)], row_v)
            pltpu.sync_copy(row_v, out_hbm.at[pl.ds(b, 1)])

    return body


def kernel(input_ids, last_hidden_state):
    B, S, D = last_hidden_state.shape
    ids = input_ids.astype(jnp.int32)
    return _pooling_kernel(B, S, D)(ids, last_hidden_state)


# speedup vs baseline: 10.5394x; 1.0055x over previous
import functools

import jax
import jax.numpy as jnp
from jax import lax
from jax.experimental import pallas as pl
from jax.experimental.pallas import tpu as pltpu
from jax.experimental.pallas import tpu_sc as plsc

_NUM_CORES = 1
_NUM_SUBCORES = 16
_LANES = 16


def _pooling_kernel(B, S, D):
    mesh = plsc.VectorSubcoreMesh(
        core_axis_name="c",
        subcore_axis_name="s",
        num_cores=_NUM_CORES,
        num_subcores=_NUM_SUBCORES,
    )
    rows_per_core = B // _NUM_CORES
    workers_per_row = _NUM_SUBCORES // rows_per_core
    chunk = S // workers_per_row
    iters = chunk // _LANES

    @functools.partial(
        pl.kernel,
        out_type=jax.ShapeDtypeStruct((B, D), jnp.float32),
        mesh=mesh,
        scratch_types=[
            pltpu.VMEM((chunk,), jnp.int32),
            pltpu.VMEM((_LANES,), jnp.int32),
            pltpu.VMEM((workers_per_row, _LANES), jnp.int32),
            pltpu.VMEM((1, D), jnp.float32),
            pltpu.VMEM_SHARED((_NUM_SUBCORES, _LANES), jnp.int32),
            pltpu.SemaphoreType.DMA,
        ],
        compiler_params=pltpu.CompilerParams(
            needs_layout_passes=False, skip_device_barrier=True
        ),
    )
    def body(
        ids_hbm, lhs_hbm, out_hbm, ids_v, acc_v, parts_v, row_v, shared, sem
    ):
        sid = lax.axis_index("s")
        cid = lax.axis_index("c")
        b = rows_per_core * cid + sid // workers_per_row
        part = sid % workers_per_row

        pltpu.sync_copy(ids_hbm.at[b, pl.ds(part * chunk, chunk)], ids_v)

        rev_pos = (
            jnp.int32(S - 1 - part * chunk) - lax.iota(jnp.int32, _LANES)
        )
        acc = jnp.full((_LANES,), jnp.iinfo(jnp.int32).min, jnp.int32)
        for i in range(iters):
            vals = ids_v[pl.ds(i * _LANES, _LANES)]
            keys = lax.shift_left(vals, jnp.int32(13)) + (
                rev_pos - jnp.int32(i * _LANES)
            )
            acc = jnp.maximum(acc, keys)

        local_best = jnp.max(acc)
        local_idx = jnp.int32(S - 1) - lax.bitwise_and(
            local_best, jnp.int32(S - 1)
        )
        prefetch = pltpu.async_copy(
            lhs_hbm.at[b, pl.ds(local_idx, 1)], row_v, sem
        )

        acc_v[...] = acc
        pltpu.sync_copy(acc_v, shared.at[sid])
        plsc.subcore_barrier()

        base = (sid // workers_per_row) * workers_per_row
        pltpu.sync_copy(shared.at[pl.ds(base, workers_per_row)], parts_v)
        red = parts_v[0, :]
        for w in range(1, workers_per_row):
            red = jnp.maximum(red, parts_v[w, :])
        best = jnp.max(red)
        prefetch.wait()

        @pl.when(best == local_best)
        def _():
            pltpu.sync_copy(row_v, out_hbm.at[pl.ds(b, 1)])

    return body


def kernel(input_ids, last_hidden_state):
    B, S, D = last_hidden_state.shape
    ids = input_ids.astype(jnp.int32)
    return _pooling_kernel(B, S, D)(ids, last_hidden_state)
